# Initial kernel scaffold; baseline (speedup 1.0000x reference)
import jax
import jax.numpy as jnp
from jax.experimental import pallas as pl


def kernel(x, edge_index, edge_attr, batch, xe1, xe2, ee1, ee2, t, mlp_w1, mlp_b1, mlp_g, mlp_bln, mlp_w2, mlp_b2, ln_g, ln_b, feat_w, feat_b, p_w1, p_b1, p_w2, p_b2):
    raise NotImplementedError("write your pallas kernel here")


# speedup vs baseline: 2.2504x; 2.2504x over previous
import functools

import jax
import jax.numpy as jnp
from jax import lax
from jax.experimental import pallas as pl
from jax.experimental.pallas import tpu as pltpu
from jax.experimental.pallas import tpu_sc as plsc

EPS = 1e-7

N = 10000
D = 128
HD = 64
K = 512
NSUB = 16
NPAD = N + 112
RZ = NPAD // NSUB


def _ln(h, g, b):
    mu = jnp.mean(h, axis=-1, keepdims=True)
    var = jnp.mean((h - mu) * (h - mu), axis=-1, keepdims=True)
    return (h - mu) * jax.lax.rsqrt(var + 1e-5) * g + b


def _make_sc_msg(n_chunks):
    mesh = plsc.VectorSubcoreMesh(core_axis_name="c", subcore_axis_name="s")

    def body(g4, srcr, ecir, dstr, tvec, zr, acc_out,
             idx, buf, sbuf, tv, sem, acc):
        c = lax.axis_index("c")
        s = lax.axis_index("s")
        pltpu.sync_copy(tvec, tv)
        tvv = tv[...]

        for p in range(2):
            q = 2 * c + p
            pltpu.sync_copy(zr.at[pl.ds(s * RZ, RZ)], acc.at[pl.ds(s * RZ, RZ)])
            plsc.subcore_barrier()

            def chunk(i, carry):
                base4 = (s * n_chunks + i) * (K // 128)
                pltpu.sync_copy(srcr.at[pl.ds(base4, 4)], idx.at[pl.ds(0, 4)])
                pltpu.sync_copy(ecir.at[pl.ds(base4, 4)], idx.at[pl.ds(4, 4)])
                pltpu.sync_copy(dstr.at[pl.ds(base4, 4)], idx.at[pl.ds(8, 4)])
                descs = []
                for j in range(4):
                    descs.append(pltpu.async_copy(
                        g4.at[q].at[idx.at[j]],
                        buf.at[pl.ds(j * 128, 128)], sem))
                for j in range(4):
                    descs.append(pltpu.async_copy(
                        g4.at[q].at[idx.at[4 + j]],
                        buf.at[pl.ds(K + j * 128, 128)], sem))
                for dsc in descs:
                    dsc.wait()

                def cbody(e, carry2):
                    for cc in range(2):
                        cs = pl.ds(cc * 16, 16)
                        a = buf[e, cs]
                        b = buf[K + e, cs]
                        m = jnp.maximum(a + b, 0.0) + EPS
                        ex = jnp.exp(m * tvv)
                        sbuf[e, cs] = m * ex
                        sbuf[e, pl.ds(32 + cc * 16, 16)] = ex
                    return carry2

                lax.fori_loop(0, K, cbody, 0)

                for j in range(4):
                    pltpu.sync_copy(sbuf.at[pl.ds(j * 128, 128)],
                                    acc.at[idx.at[8 + j]], add=True)
                return carry

            lax.fori_loop(0, n_chunks, chunk, 0)
            plsc.subcore_barrier()
            pltpu.sync_copy(acc.at[pl.ds(s * RZ, RZ)],
                            acc_out.at[q].at[pl.ds(s * RZ, RZ)])

    return pl.kernel(
        body,
        out_type=jax.ShapeDtypeStruct((4, NPAD, HD), jnp.float32),
        mesh=mesh,
        compiler_params=pltpu.CompilerParams(use_tc_tiling_on_sc=False),
        scratch_types=[
            pltpu.VMEM((12, 128), jnp.int32),
            pltpu.VMEM((2 * K, 32), jnp.float32),
            pltpu.VMEM((K, HD), jnp.float32),
            pltpu.VMEM((16,), jnp.float32),
            pltpu.SemaphoreType.DMA,
            pltpu.VMEM_SHARED((NPAD, HD), jnp.float32),
        ],
    )


def _embed_body(xr, xe1r, xe2r, lngr, lnbr, h0r, har):
    xv = xr[...]
    na = xe1r.shape[0]
    nc = xe2r.shape[0]
    oh1 = (xv[:, 0:1] == lax.broadcasted_iota(jnp.int32, (xv.shape[0], na), 1)
           ).astype(jnp.float32)
    oh2 = (xv[:, 1:2] == lax.broadcasted_iota(jnp.int32, (xv.shape[0], nc), 1)
           ).astype(jnp.float32)
    h0 = (jnp.dot(oh1, xe1r[...], preferred_element_type=jnp.float32)
          + jnp.dot(oh2, xe2r[...], preferred_element_type=jnp.float32))
    h0r[...] = h0
    har[...] = jax.nn.relu(_ln(h0, lngr[...], lnbr[...]))


def _update_body(hr, har, accr, w1r, b1r, gr, blnr, w2r, b2r,
                 lngr, lnbr, hnr, hanr):
    num = jnp.concatenate([accr[q, :, :32] for q in range(4)], axis=-1)
    den = jnp.concatenate([accr[q, :, 32:] for q in range(4)], axis=-1)
    ha = har[...]
    out = num / (den + 1e-16) + ha
    z = jnp.dot(out, w1r[...], preferred_element_type=jnp.float32) + b1r[...]
    z = jax.nn.relu(_ln(z, gr[...], blnr[...]))
    m = jnp.dot(z, w2r[...], preferred_element_type=jnp.float32) + b2r[...]
    hn = hr[...] + m
    hnr[...] = hn
    hanr[...] = jax.nn.relu(_ln(hn, lngr[...], lnbr[...]))


def _pool_body(hr, br, fwr, fbr, w1r, b1r, w2r, b2r, outr):
    ng = 64
    oh = (br[...] == lax.broadcasted_iota(jnp.int32, (N, ng), 1)
          ).astype(jnp.float32)
    sums = lax.dot_general(oh, hr[...], (((0,), (0,)), ((), ())),
                           preferred_element_type=jnp.float32)
    cnts = jnp.sum(oh, axis=0)[:, None]
    pooled = sums / jnp.maximum(cnts, 1.0)
    f = jnp.dot(pooled, fwr[...], preferred_element_type=jnp.float32) + fbr[...]
    o = jax.nn.relu(
        jnp.dot(f, w1r[...], preferred_element_type=jnp.float32) + b1r[...])
    outr[...] = jnp.dot(o, w2r[...], preferred_element_type=jnp.float32) + b2r[...]


def _full(shape):
    return pl.BlockSpec(shape, lambda i: tuple(0 for _ in shape))


def kernel(x, edge_index, edge_attr, batch, xe1, xe2, ee1, ee2, t,
           mlp_w1, mlp_b1, mlp_g, mlp_bln, mlp_w2, mlp_b2, ln_g, ln_b,
           feat_w, feat_b, p_w1, p_b1, p_w2, p_b2):
    assert x.shape == (N, 2) and xe1.shape[1] == D
    L = ee1.shape[0]
    nbd = ee2.shape[1]
    ncomb = ee1.shape[1] * nbd
    E = edge_index.shape[1]
    EP = E + N
    per_chunk = NSUB * K
    n_chunks = -(-EP // per_chunk)
    EPAD = n_chunks * per_chunk

    i32 = edge_index.dtype
    sl = jnp.arange(N, dtype=i32)
    src = jnp.concatenate([edge_index[0], sl])
    dst = jnp.concatenate([edge_index[1], sl])
    ec = jnp.concatenate([edge_attr[:, 0] * nbd + edge_attr[:, 1],
                          jnp.full((N,), 4 * nbd, dtype=i32)])
    npad_e = EPAD - EP
    src = jnp.concatenate([src, jnp.zeros((npad_e,), i32)]).reshape(-1, 128)
    dst = jnp.concatenate([dst, jnp.full((npad_e,), N, i32)]).reshape(-1, 128)
    eci = (jnp.concatenate([ec, jnp.zeros((npad_e,), i32)]) + N).reshape(-1, 128)

    comb = (ee1[:, :, None, :] + ee2[:, None, :, :]).reshape(L, ncomb, D)
    zr = jnp.zeros((NPAD, HD), jnp.float32)

    sc_msg = _make_sc_msg(n_chunks)

    R = 1000
    grid = (N // R,)
    embed = pl.pallas_call(
        _embed_body,
        grid=grid,
        in_specs=[pl.BlockSpec((R, 2), lambda i: (i, 0)),
                  _full(xe1.shape), _full(xe2.shape),
                  _full((1, D)), _full((1, D))],
        out_specs=[pl.BlockSpec((R, D), lambda i: (i, 0)),
                   pl.BlockSpec((R, D), lambda i: (i, 0))],
        out_shape=[jax.ShapeDtypeStruct((N, D), jnp.float32),
                   jax.ShapeDtypeStruct((N, D), jnp.float32)],
    )
    update = pl.pallas_call(
        _update_body,
        grid=grid,
        in_specs=[pl.BlockSpec((R, D), lambda i: (i, 0)),
                  pl.BlockSpec((R, D), lambda i: (i, 0)),
                  pl.BlockSpec((4, R, HD), lambda i: (0, i, 0)),
                  _full((D, 2 * D)), _full((1, 2 * D)), _full((1, 2 * D)),
                  _full((1, 2 * D)), _full((2 * D, D)), _full((1, D)),
                  _full((1, D)), _full((1, D))],
        out_specs=[pl.BlockSpec((R, D), lambda i: (i, 0)),
                   pl.BlockSpec((R, D), lambda i: (i, 0))],
        out_shape=[jax.ShapeDtypeStruct((N, D), jnp.float32),
                   jax.ShapeDtypeStruct((N, D), jnp.float32)],
    )
    pool = pl.pallas_call(
        _pool_body,
        out_shape=jax.ShapeDtypeStruct((64, 2), jnp.float32),
    )

    h, ha = embed(x, xe1, xe2, ln_g[0][None], ln_b[0][None])
    for l in range(L):
        g4 = jnp.stack([
            jnp.concatenate([ha[:, 32 * q:32 * q + 32],
                             comb[l][:, 32 * q:32 * q + 32]], axis=0)
            for q in range(4)])
        tvec = jnp.full((16,), t[l], jnp.float32)
        acc = sc_msg(g4, src, eci, dst, tvec, zr)
        ln_n = (l + 1) % L
        h, ha = update(h, ha, acc[:, :N],
                       mlp_w1[l], mlp_b1[l][None], mlp_g[l][None],
                       mlp_bln[l][None], mlp_w2[l], mlp_b2[l][None],
                       ln_g[ln_n][None], ln_b[ln_n][None])
    return pool(h, batch[:, None], feat_w, feat_b[None],
                p_w1, p_b1[None], p_w2, p_b2[None])


# speedup vs baseline: 2.3645x; 1.0507x over previous
import functools

import jax
import jax.numpy as jnp
from jax import lax
from jax.experimental import pallas as pl
from jax.experimental.pallas import tpu as pltpu
from jax.experimental.pallas import tpu_sc as plsc

EPS = 1e-7

N = 10000
D = 128
HD = 64
K = 512
NSUB = 16
NPAD = N + 112
RZ = NPAD // NSUB


def _ln(h, g, b):
    mu = jnp.mean(h, axis=-1, keepdims=True)
    var = jnp.mean((h - mu) * (h - mu), axis=-1, keepdims=True)
    return (h - mu) * jax.lax.rsqrt(var + 1e-5) * g + b


def _make_sc_msg(n_chunks):
    mesh = plsc.VectorSubcoreMesh(core_axis_name="c", subcore_axis_name="s")

    def body(g4, srcr, ecir, dstr, tvec, zr, acc_out,
             idx, buf, sbuf, tv, sem, acc):
        c = lax.axis_index("c")
        s = lax.axis_index("s")
        pltpu.sync_copy(tvec, tv)
        tvv = tv[...]

        for p in range(2):
            q = 2 * c + p
            pltpu.sync_copy(zr.at[pl.ds(s * RZ, RZ)], acc.at[pl.ds(s * RZ, RZ)])
            plsc.subcore_barrier()

            def chunk(i, carry):
                base4 = (s * n_chunks + i) * (K // 128)
                pltpu.sync_copy(srcr.at[pl.ds(base4, 4)], idx.at[pl.ds(0, 4)])
                pltpu.sync_copy(ecir.at[pl.ds(base4, 4)], idx.at[pl.ds(4, 4)])
                pltpu.sync_copy(dstr.at[pl.ds(base4, 4)], idx.at[pl.ds(8, 4)])
                descs = []
                for j in range(4):
                    descs.append(pltpu.async_copy(
                        g4.at[q].at[idx.at[j]],
                        buf.at[pl.ds(j * 128, 128)], sem))
                for j in range(4):
                    descs.append(pltpu.async_copy(
                        g4.at[q].at[idx.at[4 + j]],
                        buf.at[pl.ds(K + j * 128, 128)], sem))
                for dsc in descs:
                    dsc.wait()

                @plsc.parallel_loop(0, K, unroll=8)
                def cbody(e):
                    for cc in range(2):
                        cs = pl.ds(cc * 16, 16)
                        a = buf[e, cs]
                        b = buf[K + e, cs]
                        m = jnp.maximum(a + b, 0.0) + EPS
                        ex = jnp.exp(m * tvv)
                        sbuf[e, cs] = m * ex
                        sbuf[e, pl.ds(32 + cc * 16, 16)] = ex

                for j in range(4):
                    pltpu.sync_copy(sbuf.at[pl.ds(j * 128, 128)],
                                    acc.at[idx.at[8 + j]], add=True)
                return carry

            lax.fori_loop(0, n_chunks, chunk, 0)
            plsc.subcore_barrier()
            pltpu.sync_copy(acc.at[pl.ds(s * RZ, RZ)],
                            acc_out.at[q].at[pl.ds(s * RZ, RZ)])

    return pl.kernel(
        body,
        out_type=jax.ShapeDtypeStruct((4, NPAD, HD), jnp.float32),
        mesh=mesh,
        compiler_params=pltpu.CompilerParams(use_tc_tiling_on_sc=False),
        scratch_types=[
            pltpu.VMEM((12, 128), jnp.int32),
            pltpu.VMEM((2 * K, 32), jnp.float32),
            pltpu.VMEM((K, HD), jnp.float32),
            pltpu.VMEM((16,), jnp.float32),
            pltpu.SemaphoreType.DMA,
            pltpu.VMEM_SHARED((NPAD, HD), jnp.float32),
        ],
    )


def _embed_body(xr, xe1r, xe2r, lngr, lnbr, h0r, har):
    xv = xr[...]
    na = xe1r.shape[0]
    nc = xe2r.shape[0]
    oh1 = (xv[:, 0:1] == lax.broadcasted_iota(jnp.int32, (xv.shape[0], na), 1)
           ).astype(jnp.float32)
    oh2 = (xv[:, 1:2] == lax.broadcasted_iota(jnp.int32, (xv.shape[0], nc), 1)
           ).astype(jnp.float32)
    h0 = (jnp.dot(oh1, xe1r[...], preferred_element_type=jnp.float32)
          + jnp.dot(oh2, xe2r[...], preferred_element_type=jnp.float32))
    h0r[...] = h0
    har[...] = jax.nn.relu(_ln(h0, lngr[...], lnbr[...]))


def _update_body(hr, har, accr, w1r, b1r, gr, blnr, w2r, b2r,
                 lngr, lnbr, hnr, hanr):
    num = jnp.concatenate([accr[q, :, :32] for q in range(4)], axis=-1)
    den = jnp.concatenate([accr[q, :, 32:] for q in range(4)], axis=-1)
    ha = har[...]
    out = num / (den + 1e-16) + ha
    z = jnp.dot(out, w1r[...], preferred_element_type=jnp.float32) + b1r[...]
    z = jax.nn.relu(_ln(z, gr[...], blnr[...]))
    m = jnp.dot(z, w2r[...], preferred_element_type=jnp.float32) + b2r[...]
    hn = hr[...] + m
    hnr[...] = hn
    hanr[...] = jax.nn.relu(_ln(hn, lngr[...], lnbr[...]))


def _pool_body(hr, br, fwr, fbr, w1r, b1r, w2r, b2r, outr):
    ng = 64
    oh = (br[...] == lax.broadcasted_iota(jnp.int32, (N, ng), 1)
          ).astype(jnp.float32)
    sums = lax.dot_general(oh, hr[...], (((0,), (0,)), ((), ())),
                           preferred_element_type=jnp.float32)
    cnts = jnp.sum(oh, axis=0)[:, None]
    pooled = sums / jnp.maximum(cnts, 1.0)
    f = jnp.dot(pooled, fwr[...], preferred_element_type=jnp.float32) + fbr[...]
    o = jax.nn.relu(
        jnp.dot(f, w1r[...], preferred_element_type=jnp.float32) + b1r[...])
    outr[...] = jnp.dot(o, w2r[...], preferred_element_type=jnp.float32) + b2r[...]


def _full(shape):
    return pl.BlockSpec(shape, lambda i: tuple(0 for _ in shape))


def kernel(x, edge_index, edge_attr, batch, xe1, xe2, ee1, ee2, t,
           mlp_w1, mlp_b1, mlp_g, mlp_bln, mlp_w2, mlp_b2, ln_g, ln_b,
           feat_w, feat_b, p_w1, p_b1, p_w2, p_b2):
    assert x.shape == (N, 2) and xe1.shape[1] == D
    L = ee1.shape[0]
    nbd = ee2.shape[1]
    ncomb = ee1.shape[1] * nbd
    E = edge_index.shape[1]
    EP = E + N
    per_chunk = NSUB * K
    n_chunks = -(-EP // per_chunk)
    EPAD = n_chunks * per_chunk

    i32 = edge_index.dtype
    sl = jnp.arange(N, dtype=i32)
    src = jnp.concatenate([edge_index[0], sl])
    dst = jnp.concatenate([edge_index[1], sl])
    ec = jnp.concatenate([edge_attr[:, 0] * nbd + edge_attr[:, 1],
                          jnp.full((N,), 4 * nbd, dtype=i32)])
    npad_e = EPAD - EP
    src = jnp.concatenate([src, jnp.zeros((npad_e,), i32)]).reshape(-1, 128)
    dst = jnp.concatenate([dst, jnp.full((npad_e,), N, i32)]).reshape(-1, 128)
    eci = (jnp.concatenate([ec, jnp.zeros((npad_e,), i32)]) + N).reshape(-1, 128)

    comb = (ee1[:, :, None, :] + ee2[:, None, :, :]).reshape(L, ncomb, D)
    zr = jnp.zeros((NPAD, HD), jnp.float32)

    sc_msg = _make_sc_msg(n_chunks)

    R = 1000
    grid = (N // R,)
    embed = pl.pallas_call(
        _embed_body,
        grid=grid,
        in_specs=[pl.BlockSpec((R, 2), lambda i: (i, 0)),
                  _full(xe1.shape), _full(xe2.shape),
                  _full((1, D)), _full((1, D))],
        out_specs=[pl.BlockSpec((R, D), lambda i: (i, 0)),
                   pl.BlockSpec((R, D), lambda i: (i, 0))],
        out_shape=[jax.ShapeDtypeStruct((N, D), jnp.float32),
                   jax.ShapeDtypeStruct((N, D), jnp.float32)],
    )
    update = pl.pallas_call(
        _update_body,
        grid=grid,
        in_specs=[pl.BlockSpec((R, D), lambda i: (i, 0)),
                  pl.BlockSpec((R, D), lambda i: (i, 0)),
                  pl.BlockSpec((4, R, HD), lambda i: (0, i, 0)),
                  _full((D, 2 * D)), _full((1, 2 * D)), _full((1, 2 * D)),
                  _full((1, 2 * D)), _full((2 * D, D)), _full((1, D)),
                  _full((1, D)), _full((1, D))],
        out_specs=[pl.BlockSpec((R, D), lambda i: (i, 0)),
                   pl.BlockSpec((R, D), lambda i: (i, 0))],
        out_shape=[jax.ShapeDtypeStruct((N, D), jnp.float32),
                   jax.ShapeDtypeStruct((N, D), jnp.float32)],
    )
    pool = pl.pallas_call(
        _pool_body,
        out_shape=jax.ShapeDtypeStruct((64, 2), jnp.float32),
    )

    h, ha = embed(x, xe1, xe2, ln_g[0][None], ln_b[0][None])
    for l in range(L):
        g4 = jnp.stack([
            jnp.concatenate([ha[:, 32 * q:32 * q + 32],
                             comb[l][:, 32 * q:32 * q + 32]], axis=0)
            for q in range(4)])
        tvec = jnp.full((16,), t[l], jnp.float32)
        acc = sc_msg(g4, src, eci, dst, tvec, zr)
        ln_n = (l + 1) % L
        h, ha = update(h, ha, acc[:, :N],
                       mlp_w1[l], mlp_b1[l][None], mlp_g[l][None],
                       mlp_bln[l][None], mlp_w2[l], mlp_b2[l][None],
                       ln_g[ln_n][None], ln_b[ln_n][None])
    return pool(h, batch[:, None], feat_w, feat_b[None],
                p_w1, p_b1[None], p_w2, p_b2[None])


# speedup vs baseline: 2.4076x; 1.0182x over previous
import functools

import jax
import jax.numpy as jnp
from jax import lax
from jax.experimental import pallas as pl
from jax.experimental.pallas import tpu as pltpu
from jax.experimental.pallas import tpu_sc as plsc

EPS = 1e-7

N = 10000
D = 128
HD = 64
K = 384
NSUB = 16
NPAD = N + 112
RZ = NPAD // NSUB


def _ln(h, g, b):
    mu = jnp.mean(h, axis=-1, keepdims=True)
    var = jnp.mean((h - mu) * (h - mu), axis=-1, keepdims=True)
    return (h - mu) * jax.lax.rsqrt(var + 1e-5) * g + b


def _make_sc_msg(n_chunks):
    mesh = plsc.VectorSubcoreMesh(core_axis_name="c", subcore_axis_name="s")

    def body(g4, src1, eci1, dstr, tvec, zr, acc_out,
             gsrc0, gsrc1, geci0, geci1, buf0, buf1, sbuf, didx, tv,
             gsem, ssem, acc):
        c = lax.axis_index("c")
        s = lax.axis_index("s")
        pltpu.sync_copy(tvec, tv)
        tvv = tv[...]
        slots = ((gsrc0, geci0, buf0), (gsrc1, geci1, buf1))

        for p in range(2):
            q = 2 * c + p
            pltpu.sync_copy(zr.at[pl.ds(s * RZ, RZ)], acc.at[pl.ds(s * RZ, RZ)])
            plsc.subcore_barrier()

            def load_and_fire(ci, sl):
                gs, ge, buf = slots[sl]
                base = (s * n_chunks + ci) * K
                pltpu.sync_copy(src1.at[pl.ds(base, K)], gs)
                pltpu.sync_copy(eci1.at[pl.ds(base, K)], ge)
                pltpu.async_copy(g4.at[q].at[gs], buf.at[pl.ds(0, K)], gsem)
                pltpu.async_copy(g4.at[q].at[ge], buf.at[pl.ds(K, K)], gsem)

            def do_chunk(ci, sl):
                gs, ge, buf = slots[sl]
                pltpu.make_async_copy(g4.at[q].at[gs],
                                      buf.at[pl.ds(0, K)], gsem).wait()
                pltpu.make_async_copy(g4.at[q].at[ge],
                                      buf.at[pl.ds(K, K)], gsem).wait()

                @pl.when(ci + 1 < n_chunks)
                def _():
                    load_and_fire(ci + 1, 1 - sl)

                @plsc.parallel_loop(0, K, unroll=8)
                def cbody(e):
                    for cc in range(2):
                        cs = pl.ds(cc * 16, 16)
                        a = buf[e, cs]
                        b = buf[K + e, cs]
                        m = jnp.maximum(a + b, 0.0) + EPS
                        ex = jnp.exp(m * tvv)
                        sbuf[e, cs] = m * ex
                        sbuf[e, pl.ds(32 + cc * 16, 16)] = ex

                base4 = (s * n_chunks + ci) * (K // 128)
                pltpu.sync_copy(dstr.at[pl.ds(base4, K // 128)], didx)
                descs = []
                for j in range(K // 128):
                    descs.append(pltpu.async_copy(
                        sbuf.at[pl.ds(j * 128, 128)],
                        acc.at[didx.at[j]], ssem, add=True))
                for dsc in descs:
                    dsc.wait()

            load_and_fire(0, 0)

            def pair(i, carry):
                do_chunk(2 * i, 0)
                do_chunk(2 * i + 1, 1)
                return carry

            lax.fori_loop(0, n_chunks // 2, pair, 0)
            plsc.subcore_barrier()
            pltpu.sync_copy(acc.at[pl.ds(s * RZ, RZ)],
                            acc_out.at[q].at[pl.ds(s * RZ, RZ)])

    return pl.kernel(
        body,
        out_type=jax.ShapeDtypeStruct((4, NPAD, HD), jnp.float32),
        mesh=mesh,
        compiler_params=pltpu.CompilerParams(use_tc_tiling_on_sc=False),
        scratch_types=[
            pltpu.VMEM((K,), jnp.int32),
            pltpu.VMEM((K,), jnp.int32),
            pltpu.VMEM((K,), jnp.int32),
            pltpu.VMEM((K,), jnp.int32),
            pltpu.VMEM((2 * K, 32), jnp.float32),
            pltpu.VMEM((2 * K, 32), jnp.float32),
            pltpu.VMEM((K, HD), jnp.float32),
            pltpu.VMEM((K // 128, 128), jnp.int32),
            pltpu.VMEM((16,), jnp.float32),
            pltpu.SemaphoreType.DMA,
            pltpu.SemaphoreType.DMA,
            pltpu.VMEM_SHARED((NPAD, HD), jnp.float32),
        ],
    )


def _embed_body(xr, xe1r, xe2r, lngr, lnbr, h0r, har):
    xv = xr[...]
    na = xe1r.shape[0]
    nc = xe2r.shape[0]
    oh1 = (xv[:, 0:1] == lax.broadcasted_iota(jnp.int32, (xv.shape[0], na), 1)
           ).astype(jnp.float32)
    oh2 = (xv[:, 1:2] == lax.broadcasted_iota(jnp.int32, (xv.shape[0], nc), 1)
           ).astype(jnp.float32)
    h0 = (jnp.dot(oh1, xe1r[...], preferred_element_type=jnp.float32)
          + jnp.dot(oh2, xe2r[...], preferred_element_type=jnp.float32))
    h0r[...] = h0
    har[...] = jax.nn.relu(_ln(h0, lngr[...], lnbr[...]))


def _update_body(hr, har, accr, w1r, b1r, gr, blnr, w2r, b2r,
                 lngr, lnbr, hnr, hanr):
    num = jnp.concatenate([accr[q, :, :32] for q in range(4)], axis=-1)
    den = jnp.concatenate([accr[q, :, 32:] for q in range(4)], axis=-1)
    ha = har[...]
    out = num / (den + 1e-16) + ha
    z = jnp.dot(out, w1r[...], preferred_element_type=jnp.float32) + b1r[...]
    z = jax.nn.relu(_ln(z, gr[...], blnr[...]))
    m = jnp.dot(z, w2r[...], preferred_element_type=jnp.float32) + b2r[...]
    hn = hr[...] + m
    hnr[...] = hn
    hanr[...] = jax.nn.relu(_ln(hn, lngr[...], lnbr[...]))


def _pool_body(hr, br, fwr, fbr, w1r, b1r, w2r, b2r, outr):
    ng = 64
    oh = (br[...] == lax.broadcasted_iota(jnp.int32, (N, ng), 1)
          ).astype(jnp.float32)
    sums = lax.dot_general(oh, hr[...], (((0,), (0,)), ((), ())),
                           preferred_element_type=jnp.float32)
    cnts = jnp.sum(oh, axis=0)[:, None]
    pooled = sums / jnp.maximum(cnts, 1.0)
    f = jnp.dot(pooled, fwr[...], preferred_element_type=jnp.float32) + fbr[...]
    o = jax.nn.relu(
        jnp.dot(f, w1r[...], preferred_element_type=jnp.float32) + b1r[...])
    outr[...] = jnp.dot(o, w2r[...], preferred_element_type=jnp.float32) + b2r[...]


def _full(shape):
    return pl.BlockSpec(shape, lambda i: tuple(0 for _ in shape))


def kernel(x, edge_index, edge_attr, batch, xe1, xe2, ee1, ee2, t,
           mlp_w1, mlp_b1, mlp_g, mlp_bln, mlp_w2, mlp_b2, ln_g, ln_b,
           feat_w, feat_b, p_w1, p_b1, p_w2, p_b2):
    assert x.shape == (N, 2) and xe1.shape[1] == D
    L = ee1.shape[0]
    nbd = ee2.shape[1]
    ncomb = ee1.shape[1] * nbd
    E = edge_index.shape[1]
    EP = E + N
    per_chunk = NSUB * K
    n_chunks = -(-EP // per_chunk)
    n_chunks += n_chunks % 2
    EPAD = n_chunks * per_chunk

    i32 = edge_index.dtype
    sl = jnp.arange(N, dtype=i32)
    src = jnp.concatenate([edge_index[0], sl])
    dst = jnp.concatenate([edge_index[1], sl])
    ec = jnp.concatenate([edge_attr[:, 0] * nbd + edge_attr[:, 1],
                          jnp.full((N,), 4 * nbd, dtype=i32)])
    npad_e = EPAD - EP
    src = jnp.concatenate([src, jnp.zeros((npad_e,), i32)])
    dst = jnp.concatenate([dst, jnp.full((npad_e,), N, i32)]).reshape(-1, 128)
    eci = jnp.concatenate([ec, jnp.zeros((npad_e,), i32)]) + N

    comb = (ee1[:, :, None, :] + ee2[:, None, :, :]).reshape(L, ncomb, D)
    zr = jnp.zeros((NPAD, HD), jnp.float32)

    sc_msg = _make_sc_msg(n_chunks)

    R = 1000
    grid = (N // R,)
    embed = pl.pallas_call(
        _embed_body,
        grid=grid,
        in_specs=[pl.BlockSpec((R, 2), lambda i: (i, 0)),
                  _full(xe1.shape), _full(xe2.shape),
                  _full((1, D)), _full((1, D))],
        out_specs=[pl.BlockSpec((R, D), lambda i: (i, 0)),
                   pl.BlockSpec((R, D), lambda i: (i, 0))],
        out_shape=[jax.ShapeDtypeStruct((N, D), jnp.float32),
                   jax.ShapeDtypeStruct((N, D), jnp.float32)],
    )
    update = pl.pallas_call(
        _update_body,
        grid=grid,
        in_specs=[pl.BlockSpec((R, D), lambda i: (i, 0)),
                  pl.BlockSpec((R, D), lambda i: (i, 0)),
                  pl.BlockSpec((4, R, HD), lambda i: (0, i, 0)),
                  _full((D, 2 * D)), _full((1, 2 * D)), _full((1, 2 * D)),
                  _full((1, 2 * D)), _full((2 * D, D)), _full((1, D)),
                  _full((1, D)), _full((1, D))],
        out_specs=[pl.BlockSpec((R, D), lambda i: (i, 0)),
                   pl.BlockSpec((R, D), lambda i: (i, 0))],
        out_shape=[jax.ShapeDtypeStruct((N, D), jnp.float32),
                   jax.ShapeDtypeStruct((N, D), jnp.float32)],
    )
    pool = pl.pallas_call(
        _pool_body,
        out_shape=jax.ShapeDtypeStruct((64, 2), jnp.float32),
    )

    h, ha = embed(x, xe1, xe2, ln_g[0][None], ln_b[0][None])
    for l in range(L):
        g4 = jnp.stack([
            jnp.concatenate([ha[:, 32 * q:32 * q + 32],
                             comb[l][:, 32 * q:32 * q + 32]], axis=0)
            for q in range(4)])
        tvec = jnp.full((16,), t[l], jnp.float32)
        acc = sc_msg(g4, src, eci, dst, tvec, zr)
        ln_n = (l + 1) % L
        h, ha = update(h, ha, acc[:, :N],
                       mlp_w1[l], mlp_b1[l][None], mlp_g[l][None],
                       mlp_bln[l][None], mlp_w2[l], mlp_b2[l][None],
                       ln_g[ln_n][None], ln_b[ln_n][None])
    return pool(h, batch[:, None], feat_w, feat_b[None],
                p_w1, p_b1[None], p_w2, p_b2[None])


# speedup vs baseline: 7.1305x; 2.9617x over previous
import functools

import jax
import jax.numpy as jnp
from jax import lax
from jax.experimental import pallas as pl
from jax.experimental.pallas import tpu as pltpu
from jax.experimental.pallas import tpu_sc as plsc

EPS = 1e-7

N = 10000
D = 128
HD = 64
K = 512
NSUB = 16
NPAD = N + 112
RZ = NPAD // NSUB


def _ln(h, g, b):
    mu = jnp.mean(h, axis=-1, keepdims=True)
    var = jnp.mean((h - mu) * (h - mu), axis=-1, keepdims=True)
    return (h - mu) * jax.lax.rsqrt(var + 1e-5) * g + b


def _make_sc_msg(n_chunks):
    mesh = plsc.VectorSubcoreMesh(core_axis_name="c", subcore_axis_name="s")

    def body(g4, src1, dstr, combf, tvec, zr, acc_out,
             gsrc0, gsrc1, buf0, buf1, cbuf0, cbuf1, sbuf, didx, tv,
             gsem, ssem, acc):
        c = lax.axis_index("c")
        s = lax.axis_index("s")
        pltpu.sync_copy(tvec, tv)
        tvv = tv[...]
        slots = ((gsrc0, buf0, cbuf0), (gsrc1, buf1, cbuf1))

        for p in range(2):
            q = 2 * c + p
            pltpu.sync_copy(zr.at[pl.ds(s * RZ, RZ)], acc.at[pl.ds(s * RZ, RZ)])
            plsc.subcore_barrier()

            def load_and_fire(ci, sl):
                gs, buf, cb = slots[sl]
                g = s * n_chunks + ci
                pltpu.sync_copy(src1.at[pl.ds(g * K, K)], gs)
                pltpu.sync_copy(combf.at[q].at[pl.ds(g * 32, 32)], cb)
                pltpu.async_copy(g4.at[q].at[gs], buf, gsem)

            def do_chunk(ci, sl):
                gs, buf, cb = slots[sl]
                pltpu.make_async_copy(g4.at[q].at[gs], buf, gsem).wait()

                @pl.when(ci + 1 < n_chunks)
                def _():
                    load_and_fire(ci + 1, 1 - sl)

                @plsc.parallel_loop(0, K, unroll=8)
                def cbody(e):
                    for cc in range(2):
                        cs = pl.ds(cc * 16, 16)
                        a = buf[e, cs]
                        b = cb[cs]
                        m = jnp.maximum(a + b, 0.0) + EPS
                        ex = jnp.exp(m * tvv)
                        sbuf[e, cs] = m * ex
                        sbuf[e, pl.ds(32 + cc * 16, 16)] = ex

                base4 = (s * n_chunks + ci) * (K // 128)
                pltpu.sync_copy(dstr.at[pl.ds(base4, K // 128)], didx)
                descs = []
                for j in range(K // 128):
                    descs.append(pltpu.async_copy(
                        sbuf.at[pl.ds(j * 128, 128)],
                        acc.at[didx.at[j]], ssem, add=True))
                for dsc in descs:
                    dsc.wait()

            load_and_fire(0, 0)

            def pair(i, carry):
                do_chunk(2 * i, 0)
                do_chunk(2 * i + 1, 1)
                return carry

            lax.fori_loop(0, n_chunks // 2, pair, 0)
            plsc.subcore_barrier()
            pltpu.sync_copy(acc.at[pl.ds(s * RZ, RZ)],
                            acc_out.at[q].at[pl.ds(s * RZ, RZ)])

    return pl.kernel(
        body,
        out_type=jax.ShapeDtypeStruct((4, NPAD, HD), jnp.float32),
        mesh=mesh,
        compiler_params=pltpu.CompilerParams(use_tc_tiling_on_sc=False),
        scratch_types=[
            pltpu.VMEM((K,), jnp.int32),
            pltpu.VMEM((K,), jnp.int32),
            pltpu.VMEM((K, 32), jnp.float32),
            pltpu.VMEM((K, 32), jnp.float32),
            pltpu.VMEM((32,), jnp.float32),
            pltpu.VMEM((32,), jnp.float32),
            pltpu.VMEM((K, HD), jnp.float32),
            pltpu.VMEM((K // 128, 128), jnp.int32),
            pltpu.VMEM((16,), jnp.float32),
            pltpu.SemaphoreType.DMA,
            pltpu.SemaphoreType.DMA,
            pltpu.VMEM_SHARED((NPAD, HD), jnp.float32),
        ],
    )


def _embed_body(xr, xe1r, xe2r, lngr, lnbr, h0r, har):
    xv = xr[...]
    na = xe1r.shape[0]
    nc = xe2r.shape[0]
    oh1 = (xv[:, 0:1] == lax.broadcasted_iota(jnp.int32, (xv.shape[0], na), 1)
           ).astype(jnp.float32)
    oh2 = (xv[:, 1:2] == lax.broadcasted_iota(jnp.int32, (xv.shape[0], nc), 1)
           ).astype(jnp.float32)
    h0 = (jnp.dot(oh1, xe1r[...], preferred_element_type=jnp.float32)
          + jnp.dot(oh2, xe2r[...], preferred_element_type=jnp.float32))
    h0r[...] = h0
    har[...] = jax.nn.relu(_ln(h0, lngr[...], lnbr[...]))


def _update_body(hr, har, accr, w1r, b1r, gr, blnr, w2r, b2r,
                 lngr, lnbr, hnr, hanr):
    num = jnp.concatenate([accr[q, :, :32] for q in range(4)], axis=-1)
    den = jnp.concatenate([accr[q, :, 32:] for q in range(4)], axis=-1)
    ha = har[...]
    out = num / (den + 1e-16) + ha
    z = jnp.dot(out, w1r[...], preferred_element_type=jnp.float32) + b1r[...]
    z = jax.nn.relu(_ln(z, gr[...], blnr[...]))
    m = jnp.dot(z, w2r[...], preferred_element_type=jnp.float32) + b2r[...]
    hn = hr[...] + m
    hnr[...] = hn
    hanr[...] = jax.nn.relu(_ln(hn, lngr[...], lnbr[...]))


def _pool_body(hr, br, fwr, fbr, w1r, b1r, w2r, b2r, outr):
    ng = 64
    oh = (br[...] == lax.broadcasted_iota(jnp.int32, (N, ng), 1)
          ).astype(jnp.float32)
    sums = lax.dot_general(oh, hr[...], (((0,), (0,)), ((), ())),
                           preferred_element_type=jnp.float32)
    cnts = jnp.sum(oh, axis=0)[:, None]
    pooled = sums / jnp.maximum(cnts, 1.0)
    f = jnp.dot(pooled, fwr[...], preferred_element_type=jnp.float32) + fbr[...]
    o = jax.nn.relu(
        jnp.dot(f, w1r[...], preferred_element_type=jnp.float32) + b1r[...])
    outr[...] = jnp.dot(o, w2r[...], preferred_element_type=jnp.float32) + b2r[...]


def _full(shape):
    return pl.BlockSpec(shape, lambda i: tuple(0 for _ in shape))


def kernel(x, edge_index, edge_attr, batch, xe1, xe2, ee1, ee2, t,
           mlp_w1, mlp_b1, mlp_g, mlp_bln, mlp_w2, mlp_b2, ln_g, ln_b,
           feat_w, feat_b, p_w1, p_b1, p_w2, p_b2):
    assert x.shape == (N, 2) and xe1.shape[1] == D
    L = ee1.shape[0]
    nbd = ee2.shape[1]
    ncomb = ee1.shape[1] * nbd
    E = edge_index.shape[1]
    EP = E + N
    per2 = 2 * NSUB * K
    EPAD = -(-(EP + ncomb * (K - 1)) // per2) * per2
    n_chunks = EPAD // (NSUB * K)
    TCH = EPAD // K

    i32 = edge_index.dtype
    sl = jnp.arange(N, dtype=i32)
    src0 = jnp.concatenate([edge_index[0], sl])
    dst0 = jnp.concatenate([edge_index[1], sl])
    ec0 = jnp.concatenate([edge_attr[:, 0] * nbd + edge_attr[:, 1],
                           jnp.full((N,), 4 * nbd, dtype=i32)])
    order = jnp.argsort(ec0, stable=True)
    cnt = jnp.bincount(ec0, length=ncomb)
    pc = -(-cnt // K) * K
    cum_pc = jnp.cumsum(pc)
    offs = cum_pc - pc
    starts = jnp.cumsum(cnt) - cnt
    sec = ec0[order]
    dest = offs[sec] + jnp.arange(EP, dtype=i32) - starts[sec]
    src = (jnp.arange(EPAD, dtype=i32) % N).at[dest].set(src0[order])
    dst = (N + jnp.arange(EPAD, dtype=i32) % (NPAD - N)).at[dest].set(
        dst0[order]).reshape(-1, 128)
    chunk_ec = jnp.clip(
        jnp.searchsorted(cum_pc, jnp.arange(TCH) * K, side="right"),
        0, ncomb - 1)

    comb = (ee1[:, :, None, :] + ee2[:, None, :, :]).reshape(L, ncomb, D)
    zr = jnp.zeros((NPAD, HD), jnp.float32)

    sc_msg = _make_sc_msg(n_chunks)

    R = 1000
    grid = (N // R,)
    embed = pl.pallas_call(
        _embed_body,
        grid=grid,
        in_specs=[pl.BlockSpec((R, 2), lambda i: (i, 0)),
                  _full(xe1.shape), _full(xe2.shape),
                  _full((1, D)), _full((1, D))],
        out_specs=[pl.BlockSpec((R, D), lambda i: (i, 0)),
                   pl.BlockSpec((R, D), lambda i: (i, 0))],
        out_shape=[jax.ShapeDtypeStruct((N, D), jnp.float32),
                   jax.ShapeDtypeStruct((N, D), jnp.float32)],
    )
    update = pl.pallas_call(
        _update_body,
        grid=grid,
        in_specs=[pl.BlockSpec((R, D), lambda i: (i, 0)),
                  pl.BlockSpec((R, D), lambda i: (i, 0)),
                  pl.BlockSpec((4, R, HD), lambda i: (0, i, 0)),
                  _full((D, 2 * D)), _full((1, 2 * D)), _full((1, 2 * D)),
                  _full((1, 2 * D)), _full((2 * D, D)), _full((1, D)),
                  _full((1, D)), _full((1, D))],
        out_specs=[pl.BlockSpec((R, D), lambda i: (i, 0)),
                   pl.BlockSpec((R, D), lambda i: (i, 0))],
        out_shape=[jax.ShapeDtypeStruct((N, D), jnp.float32),
                   jax.ShapeDtypeStruct((N, D), jnp.float32)],
    )
    pool = pl.pallas_call(
        _pool_body,
        out_shape=jax.ShapeDtypeStruct((64, 2), jnp.float32),
    )

    h, ha = embed(x, xe1, xe2, ln_g[0][None], ln_b[0][None])
    for l in range(L):
        g4 = jnp.stack([ha[:, 32 * q:32 * q + 32] for q in range(4)])
        crows = comb[l][chunk_ec]
        combf = jnp.stack([crows[:, 32 * q:32 * q + 32].reshape(-1)
                           for q in range(4)])
        tvec = jnp.full((16,), t[l], jnp.float32)
        acc = sc_msg(g4, src, dst, combf, tvec, zr)
        ln_n = (l + 1) % L
        h, ha = update(h, ha, acc[:, :N],
                       mlp_w1[l], mlp_b1[l][None], mlp_g[l][None],
                       mlp_bln[l][None], mlp_w2[l], mlp_b2[l][None],
                       ln_g[ln_n][None], ln_b[ln_n][None])
    return pool(h, batch[:, None], feat_w, feat_b[None],
                p_w1, p_b1[None], p_w2, p_b2[None])


# speedup vs baseline: 7.4513x; 1.0450x over previous
import functools

import jax
import jax.numpy as jnp
from jax import lax
from jax.experimental import pallas as pl
from jax.experimental.pallas import tpu as pltpu
from jax.experimental.pallas import tpu_sc as plsc

EPS = 1e-7

N = 10000
D = 128
HD = 64
K = 512
NSUB = 16
NPAD = N + 112
RZ = NPAD // NSUB


def _ln(h, g, b):
    mu = jnp.mean(h, axis=-1, keepdims=True)
    var = jnp.mean((h - mu) * (h - mu), axis=-1, keepdims=True)
    return (h - mu) * jax.lax.rsqrt(var + 1e-5) * g + b


def _make_sc_msg(n_chunks):
    mesh = plsc.VectorSubcoreMesh(core_axis_name="c", subcore_axis_name="s")

    def body(g4, src1, dstr, combf, tvec, zr, acc_out,
             gsrc0, gsrc1, buf0, buf1, cbuf0, cbuf1, sbuf, didx, tv,
             gsem, ssem, acc):
        c = lax.axis_index("c")
        s = lax.axis_index("s")
        pltpu.sync_copy(tvec, tv)
        tvv = tv[...]
        slots = ((gsrc0, buf0, cbuf0), (gsrc1, buf1, cbuf1))

        for p in range(2):
            q = 2 * c + p
            pltpu.sync_copy(zr.at[pl.ds(s * RZ, RZ)], acc.at[pl.ds(s * RZ, RZ)])
            plsc.subcore_barrier()

            def load_and_fire(ci, sl):
                gs, buf, cb = slots[sl]
                g = s * n_chunks + ci
                pltpu.sync_copy(src1.at[pl.ds(g * K, K)], gs)
                pltpu.sync_copy(combf.at[q].at[pl.ds(g * 32, 32)], cb)
                pltpu.async_copy(g4.at[q].at[gs], buf, gsem)

            def do_chunk(ci, sl):
                gs, buf, cb = slots[sl]
                pltpu.make_async_copy(g4.at[q].at[gs], buf, gsem).wait()

                @pl.when(ci + 1 < n_chunks)
                def _():
                    load_and_fire(ci + 1, 1 - sl)

                @plsc.parallel_loop(0, K, unroll=8)
                def cbody(e):
                    for cc in range(2):
                        cs = pl.ds(cc * 16, 16)
                        a = buf[e, cs]
                        b = cb[cs]
                        m = jnp.maximum(a + b, 0.0) + EPS
                        ex = jnp.exp(m * tvv)
                        sbuf[e, cs] = m * ex
                        sbuf[e, pl.ds(32 + cc * 16, 16)] = ex

                base4 = (s * n_chunks + ci) * (K // 128)
                pltpu.sync_copy(dstr.at[pl.ds(base4, K // 128)], didx)
                descs = []
                for j in range(K // 128):
                    descs.append(pltpu.async_copy(
                        sbuf.at[pl.ds(j * 128, 128)],
                        acc.at[didx.at[j]], ssem, add=True))
                for dsc in descs:
                    dsc.wait()

            load_and_fire(0, 0)

            def pair(i, carry):
                do_chunk(2 * i, 0)
                do_chunk(2 * i + 1, 1)
                return carry

            lax.fori_loop(0, n_chunks // 2, pair, 0)
            plsc.subcore_barrier()
            pltpu.sync_copy(acc.at[pl.ds(s * RZ, RZ)],
                            acc_out.at[q].at[pl.ds(s * RZ, RZ)])

    return pl.kernel(
        body,
        out_type=jax.ShapeDtypeStruct((4, NPAD, HD), jnp.float32),
        mesh=mesh,
        compiler_params=pltpu.CompilerParams(use_tc_tiling_on_sc=False),
        scratch_types=[
            pltpu.VMEM((K,), jnp.int32),
            pltpu.VMEM((K,), jnp.int32),
            pltpu.VMEM((K, 32), jnp.float32),
            pltpu.VMEM((K, 32), jnp.float32),
            pltpu.VMEM((32,), jnp.float32),
            pltpu.VMEM((32,), jnp.float32),
            pltpu.VMEM((K, HD), jnp.float32),
            pltpu.VMEM((K // 128, 128), jnp.int32),
            pltpu.VMEM((16,), jnp.float32),
            pltpu.SemaphoreType.DMA,
            pltpu.SemaphoreType.DMA,
            pltpu.VMEM_SHARED((NPAD, HD), jnp.float32),
        ],
    )


def _embed_body(xr, xe1r, xe2r, lngr, lnbr, h0r, har):
    xv = xr[...]
    na = xe1r.shape[0]
    nc = xe2r.shape[0]
    oh1 = (xv[:, 0:1] == lax.broadcasted_iota(jnp.int32, (xv.shape[0], na), 1)
           ).astype(jnp.float32)
    oh2 = (xv[:, 1:2] == lax.broadcasted_iota(jnp.int32, (xv.shape[0], nc), 1)
           ).astype(jnp.float32)
    h0 = (jnp.dot(oh1, xe1r[...], preferred_element_type=jnp.float32)
          + jnp.dot(oh2, xe2r[...], preferred_element_type=jnp.float32))
    h0r[...] = h0
    har[...] = jax.nn.relu(_ln(h0, lngr[...], lnbr[...]))


def _update_body(hr, har, accr, w1r, b1r, gr, blnr, w2r, b2r,
                 lngr, lnbr, hnr, hanr):
    num = jnp.concatenate([accr[q, :, :32] for q in range(4)], axis=-1)
    den = jnp.concatenate([accr[q, :, 32:] for q in range(4)], axis=-1)
    ha = har[...]
    out = num / (den + 1e-16) + ha
    z = jnp.dot(out, w1r[...], preferred_element_type=jnp.float32) + b1r[...]
    z = jax.nn.relu(_ln(z, gr[...], blnr[...]))
    m = jnp.dot(z, w2r[...], preferred_element_type=jnp.float32) + b2r[...]
    hn = hr[...] + m
    hnr[...] = hn
    hanr[...] = jax.nn.relu(_ln(hn, lngr[...], lnbr[...]))


def _pool_body(hr, br, fwr, fbr, w1r, b1r, w2r, b2r, outr):
    ng = 64
    oh = (br[...] == lax.broadcasted_iota(jnp.int32, (N, ng), 1)
          ).astype(jnp.float32)
    sums = lax.dot_general(oh, hr[...], (((0,), (0,)), ((), ())),
                           preferred_element_type=jnp.float32)
    cnts = jnp.sum(oh, axis=0)[:, None]
    pooled = sums / jnp.maximum(cnts, 1.0)
    f = jnp.dot(pooled, fwr[...], preferred_element_type=jnp.float32) + fbr[...]
    o = jax.nn.relu(
        jnp.dot(f, w1r[...], preferred_element_type=jnp.float32) + b1r[...])
    outr[...] = jnp.dot(o, w2r[...], preferred_element_type=jnp.float32) + b2r[...]


def _full(shape):
    return pl.BlockSpec(shape, lambda i: tuple(0 for _ in shape))


def kernel(x, edge_index, edge_attr, batch, xe1, xe2, ee1, ee2, t,
           mlp_w1, mlp_b1, mlp_g, mlp_bln, mlp_w2, mlp_b2, ln_g, ln_b,
           feat_w, feat_b, p_w1, p_b1, p_w2, p_b2):
    assert x.shape == (N, 2) and xe1.shape[1] == D
    L = ee1.shape[0]
    nbd = ee2.shape[1]
    ncomb = ee1.shape[1] * nbd
    E = edge_index.shape[1]
    EP = E + N
    per2 = 2 * NSUB * K
    EPAD = -(-(EP + ncomb * (K - 1)) // per2) * per2
    n_chunks = EPAD // (NSUB * K)
    TCH = EPAD // K

    i32 = edge_index.dtype
    sl = jnp.arange(N, dtype=i32)
    src0 = jnp.concatenate([edge_index[0], sl])
    dst0 = jnp.concatenate([edge_index[1], sl])
    ec0 = jnp.concatenate([edge_attr[:, 0] * nbd + edge_attr[:, 1],
                           jnp.full((N,), 4 * nbd, dtype=i32)])
    oh = (ec0[:, None] == jnp.arange(ncomb, dtype=i32)[None, :]).astype(i32)
    csum = jnp.cumsum(oh, axis=0)
    rank = jnp.take_along_axis(csum, ec0[:, None].astype(jnp.int32),
                               axis=1)[:, 0] - 1
    cnt = csum[-1]
    pc = -(-cnt // K) * K
    cum_pc = jnp.cumsum(pc)
    offs = cum_pc - pc
    dest = offs[ec0] + rank
    src = (jnp.arange(EPAD, dtype=i32) % N).at[dest].set(src0)
    dst = (N + jnp.arange(EPAD, dtype=i32) % (NPAD - N)).at[dest].set(
        dst0).reshape(-1, 128)
    chunk_ec = jnp.clip(
        jnp.searchsorted(cum_pc, jnp.arange(TCH) * K, side="right"),
        0, ncomb - 1)

    comb = (ee1[:, :, None, :] + ee2[:, None, :, :]).reshape(L, ncomb, D)
    zr = jnp.zeros((NPAD, HD), jnp.float32)

    sc_msg = _make_sc_msg(n_chunks)

    R = 1000
    grid = (N // R,)
    embed = pl.pallas_call(
        _embed_body,
        grid=grid,
        in_specs=[pl.BlockSpec((R, 2), lambda i: (i, 0)),
                  _full(xe1.shape), _full(xe2.shape),
                  _full((1, D)), _full((1, D))],
        out_specs=[pl.BlockSpec((R, D), lambda i: (i, 0)),
                   pl.BlockSpec((R, D), lambda i: (i, 0))],
        out_shape=[jax.ShapeDtypeStruct((N, D), jnp.float32),
                   jax.ShapeDtypeStruct((N, D), jnp.float32)],
    )
    update = pl.pallas_call(
        _update_body,
        grid=grid,
        in_specs=[pl.BlockSpec((R, D), lambda i: (i, 0)),
                  pl.BlockSpec((R, D), lambda i: (i, 0)),
                  pl.BlockSpec((4, R, HD), lambda i: (0, i, 0)),
                  _full((D, 2 * D)), _full((1, 2 * D)), _full((1, 2 * D)),
                  _full((1, 2 * D)), _full((2 * D, D)), _full((1, D)),
                  _full((1, D)), _full((1, D))],
        out_specs=[pl.BlockSpec((R, D), lambda i: (i, 0)),
                   pl.BlockSpec((R, D), lambda i: (i, 0))],
        out_shape=[jax.ShapeDtypeStruct((N, D), jnp.float32),
                   jax.ShapeDtypeStruct((N, D), jnp.float32)],
    )
    pool = pl.pallas_call(
        _pool_body,
        out_shape=jax.ShapeDtypeStruct((64, 2), jnp.float32),
    )

    h, ha = embed(x, xe1, xe2, ln_g[0][None], ln_b[0][None])
    for l in range(L):
        g4 = jnp.stack([ha[:, 32 * q:32 * q + 32] for q in range(4)])
        crows = comb[l][chunk_ec]
        combf = jnp.stack([crows[:, 32 * q:32 * q + 32].reshape(-1)
                           for q in range(4)])
        tvec = jnp.full((16,), t[l], jnp.float32)
        acc = sc_msg(g4, src, dst, combf, tvec, zr)
        ln_n = (l + 1) % L
        h, ha = update(h, ha, acc[:, :N],
                       mlp_w1[l], mlp_b1[l][None], mlp_g[l][None],
                       mlp_bln[l][None], mlp_w2[l], mlp_b2[l][None],
                       ln_g[ln_n][None], ln_b[ln_n][None])
    return pool(h, batch[:, None], feat_w, feat_b[None],
                p_w1, p_b1[None], p_w2, p_b2[None])


# speedup vs baseline: 12.2368x; 1.6422x over previous
import functools

import jax
import jax.numpy as jnp
from jax import lax
from jax.experimental import pallas as pl
from jax.experimental.pallas import tpu as pltpu
from jax.experimental.pallas import tpu_sc as plsc

EPS = 1e-7

N = 10000
D = 128
HD = 64
K = 512
NSUB = 16
NPAD = N + 112
RZ = NPAD // NSUB


def _ln(h, g, b):
    mu = jnp.mean(h, axis=-1, keepdims=True)
    var = jnp.mean((h - mu) * (h - mu), axis=-1, keepdims=True)
    return (h - mu) * jax.lax.rsqrt(var + 1e-5) * g + b


def _make_sc_msg(n_chunks):
    mesh = plsc.VectorSubcoreMesh(core_axis_name="c", subcore_axis_name="s")

    def body(g4, src1, dstr, combf, tvec, zr, acc_out,
             gsrc0, gsrc1, buf0, buf1, cbuf0, cbuf1, sbuf, didx, tv,
             gsem, ssem, acc):
        c = lax.axis_index("c")
        s = lax.axis_index("s")
        pltpu.sync_copy(tvec, tv)
        tvv = tv[...]
        slots = ((gsrc0, buf0, cbuf0), (gsrc1, buf1, cbuf1))

        for p in range(2):
            q = 2 * c + p
            pltpu.sync_copy(zr.at[pl.ds(s * RZ, RZ)], acc.at[pl.ds(s * RZ, RZ)])
            plsc.subcore_barrier()

            def load_and_fire(ci, sl):
                gs, buf, cb = slots[sl]
                g = s * n_chunks + ci
                pltpu.sync_copy(src1.at[pl.ds(g * K, K)], gs)
                pltpu.sync_copy(combf.at[q].at[pl.ds(g * 32, 32)], cb)
                pltpu.async_copy(g4.at[q].at[gs], buf, gsem)

            def do_chunk(ci, sl):
                gs, buf, cb = slots[sl]
                pltpu.make_async_copy(g4.at[q].at[gs], buf, gsem).wait()

                @pl.when(ci + 1 < n_chunks)
                def _():
                    load_and_fire(ci + 1, 1 - sl)

                @plsc.parallel_loop(0, K, unroll=8)
                def cbody(e):
                    for cc in range(2):
                        cs = pl.ds(cc * 16, 16)
                        a = buf[e, cs]
                        b = cb[cs]
                        m = jnp.maximum(a + b, 0.0) + EPS
                        ex = jnp.exp(m * tvv)
                        sbuf[e, cs] = m * ex
                        sbuf[e, pl.ds(32 + cc * 16, 16)] = ex

                base4 = (s * n_chunks + ci) * (K // 128)
                pltpu.sync_copy(dstr.at[pl.ds(base4, K // 128)], didx)
                descs = []
                for j in range(K // 128):
                    descs.append(pltpu.async_copy(
                        sbuf.at[pl.ds(j * 128, 128)],
                        acc.at[didx.at[j]], ssem, add=True))
                for dsc in descs:
                    dsc.wait()

            load_and_fire(0, 0)

            def pair(i, carry):
                do_chunk(2 * i, 0)
                do_chunk(2 * i + 1, 1)
                return carry

            lax.fori_loop(0, n_chunks // 2, pair, 0)
            plsc.subcore_barrier()
            pltpu.sync_copy(acc.at[pl.ds(s * RZ, RZ)],
                            acc_out.at[q].at[pl.ds(s * RZ, RZ)])

    return pl.kernel(
        body,
        out_type=jax.ShapeDtypeStruct((4, NPAD, HD), jnp.float32),
        mesh=mesh,
        compiler_params=pltpu.CompilerParams(use_tc_tiling_on_sc=False),
        scratch_types=[
            pltpu.VMEM((K,), jnp.int32),
            pltpu.VMEM((K,), jnp.int32),
            pltpu.VMEM((K, 32), jnp.float32),
            pltpu.VMEM((K, 32), jnp.float32),
            pltpu.VMEM((32,), jnp.float32),
            pltpu.VMEM((32,), jnp.float32),
            pltpu.VMEM((K, HD), jnp.float32),
            pltpu.VMEM((K // 128, 128), jnp.int32),
            pltpu.VMEM((16,), jnp.float32),
            pltpu.SemaphoreType.DMA,
            pltpu.SemaphoreType.DMA,
            pltpu.VMEM_SHARED((NPAD, HD), jnp.float32),
        ],
    )


def _embed_body(xr, xe1r, xe2r, lngr, lnbr, h0r, har):
    xv = xr[...]
    na = xe1r.shape[0]
    nc = xe2r.shape[0]
    oh1 = (xv[:, 0:1] == lax.broadcasted_iota(jnp.int32, (xv.shape[0], na), 1)
           ).astype(jnp.float32)
    oh2 = (xv[:, 1:2] == lax.broadcasted_iota(jnp.int32, (xv.shape[0], nc), 1)
           ).astype(jnp.float32)
    h0 = (jnp.dot(oh1, xe1r[...], preferred_element_type=jnp.float32)
          + jnp.dot(oh2, xe2r[...], preferred_element_type=jnp.float32))
    h0r[...] = h0
    har[...] = jax.nn.relu(_ln(h0, lngr[...], lnbr[...]))


def _update_body(hr, har, accr, w1r, b1r, gr, blnr, w2r, b2r,
                 lngr, lnbr, hnr, hanr):
    num = jnp.concatenate([accr[q, :, :32] for q in range(4)], axis=-1)
    den = jnp.concatenate([accr[q, :, 32:] for q in range(4)], axis=-1)
    ha = har[...]
    out = num / (den + 1e-16) + ha
    z = jnp.dot(out, w1r[...], preferred_element_type=jnp.float32) + b1r[...]
    z = jax.nn.relu(_ln(z, gr[...], blnr[...]))
    m = jnp.dot(z, w2r[...], preferred_element_type=jnp.float32) + b2r[...]
    hn = hr[...] + m
    hnr[...] = hn
    hanr[...] = jax.nn.relu(_ln(hn, lngr[...], lnbr[...]))


def _pool_body(hr, br, fwr, fbr, w1r, b1r, w2r, b2r, outr):
    ng = 64
    oh = (br[...] == lax.broadcasted_iota(jnp.int32, (N, ng), 1)
          ).astype(jnp.float32)
    sums = lax.dot_general(oh, hr[...], (((0,), (0,)), ((), ())),
                           preferred_element_type=jnp.float32)
    cnts = jnp.sum(oh, axis=0)[:, None]
    pooled = sums / jnp.maximum(cnts, 1.0)
    f = jnp.dot(pooled, fwr[...], preferred_element_type=jnp.float32) + fbr[...]
    o = jax.nn.relu(
        jnp.dot(f, w1r[...], preferred_element_type=jnp.float32) + b1r[...])
    outr[...] = jnp.dot(o, w2r[...], preferred_element_type=jnp.float32) + b2r[...]


def _full(shape):
    return pl.BlockSpec(shape, lambda i: tuple(0 for _ in shape))


def kernel(x, edge_index, edge_attr, batch, xe1, xe2, ee1, ee2, t,
           mlp_w1, mlp_b1, mlp_g, mlp_bln, mlp_w2, mlp_b2, ln_g, ln_b,
           feat_w, feat_b, p_w1, p_b1, p_w2, p_b2):
    assert x.shape == (N, 2) and xe1.shape[1] == D
    L = ee1.shape[0]
    nbd = ee2.shape[1]
    ncomb = ee1.shape[1] * nbd
    E = edge_index.shape[1]
    EP = E + N
    per2 = 2 * NSUB * K
    EPAD = -(-(EP + ncomb * (K - 1)) // per2) * per2
    n_chunks = EPAD // (NSUB * K)
    TCH = EPAD // K

    i32 = edge_index.dtype
    sl = jnp.arange(N, dtype=i32)
    src0 = jnp.concatenate([edge_index[0], sl])
    dst0 = jnp.concatenate([edge_index[1], sl])
    ec0 = jnp.concatenate([edge_attr[:, 0] * nbd + edge_attr[:, 1],
                           jnp.full((N,), 4 * nbd, dtype=i32)])
    sec, s_src, s_dst = lax.sort([ec0, src0, dst0], num_keys=1)
    cnt = jnp.sum(
        (ec0[:, None] == jnp.arange(ncomb, dtype=i32)[None, :]).astype(i32),
        axis=0)
    pc = -(-cnt // K) * K
    cum_pc = jnp.cumsum(pc)
    offs = cum_pc - pc
    starts = jnp.cumsum(cnt) - cnt
    i = jnp.arange(EPAD, dtype=i32)
    b = jnp.clip(jnp.sum((i[:, None] >= cum_pc[None, :]).astype(i32), axis=1),
                 0, ncomb - 1)
    r = i - offs[b]
    valid = r < cnt[b]
    si = jnp.clip(starts[b] + r, 0, EP - 1)
    src = jnp.where(valid, s_src[si], i % N)
    dst = jnp.where(valid, s_dst[si],
                    N + i % (NPAD - N)).reshape(-1, 128)
    chunk_ec = b[jnp.arange(TCH) * K]

    comb = (ee1[:, :, None, :] + ee2[:, None, :, :]).reshape(L, ncomb, D)
    zr = jnp.zeros((NPAD, HD), jnp.float32)

    sc_msg = _make_sc_msg(n_chunks)

    R = 1000
    grid = (N // R,)
    embed = pl.pallas_call(
        _embed_body,
        grid=grid,
        in_specs=[pl.BlockSpec((R, 2), lambda i: (i, 0)),
                  _full(xe1.shape), _full(xe2.shape),
                  _full((1, D)), _full((1, D))],
        out_specs=[pl.BlockSpec((R, D), lambda i: (i, 0)),
                   pl.BlockSpec((R, D), lambda i: (i, 0))],
        out_shape=[jax.ShapeDtypeStruct((N, D), jnp.float32),
                   jax.ShapeDtypeStruct((N, D), jnp.float32)],
    )
    update = pl.pallas_call(
        _update_body,
        grid=grid,
        in_specs=[pl.BlockSpec((R, D), lambda i: (i, 0)),
                  pl.BlockSpec((R, D), lambda i: (i, 0)),
                  pl.BlockSpec((4, R, HD), lambda i: (0, i, 0)),
                  _full((D, 2 * D)), _full((1, 2 * D)), _full((1, 2 * D)),
                  _full((1, 2 * D)), _full((2 * D, D)), _full((1, D)),
                  _full((1, D)), _full((1, D))],
        out_specs=[pl.BlockSpec((R, D), lambda i: (i, 0)),
                   pl.BlockSpec((R, D), lambda i: (i, 0))],
        out_shape=[jax.ShapeDtypeStruct((N, D), jnp.float32),
                   jax.ShapeDtypeStruct((N, D), jnp.float32)],
    )
    pool = pl.pallas_call(
        _pool_body,
        out_shape=jax.ShapeDtypeStruct((64, 2), jnp.float32),
    )

    h, ha = embed(x, xe1, xe2, ln_g[0][None], ln_b[0][None])
    for l in range(L):
        g4 = jnp.stack([ha[:, 32 * q:32 * q + 32] for q in range(4)])
        crows = comb[l][chunk_ec]
        combf = jnp.stack([crows[:, 32 * q:32 * q + 32].reshape(-1)
                           for q in range(4)])
        tvec = jnp.full((16,), t[l], jnp.float32)
        acc = sc_msg(g4, src, dst, combf, tvec, zr)
        ln_n = (l + 1) % L
        h, ha = update(h, ha, acc[:, :N],
                       mlp_w1[l], mlp_b1[l][None], mlp_g[l][None],
                       mlp_bln[l][None], mlp_w2[l], mlp_b2[l][None],
                       ln_g[ln_n][None], ln_b[ln_n][None])
    return pool(h, batch[:, None], feat_w, feat_b[None],
                p_w1, p_b1[None], p_w2, p_b2[None])


# speedup vs baseline: 13.3345x; 1.0897x over previous
import functools

import jax
import jax.numpy as jnp
from jax import lax
from jax.experimental import pallas as pl
from jax.experimental.pallas import tpu as pltpu
from jax.experimental.pallas import tpu_sc as plsc

EPS = 1e-7

N = 10000
D = 128
HD = 64
K = 512
NSUB = 16
NPAD = N + 112
RZ = NPAD // NSUB


def _ln(h, g, b):
    mu = jnp.mean(h, axis=-1, keepdims=True)
    var = jnp.mean((h - mu) * (h - mu), axis=-1, keepdims=True)
    return (h - mu) * jax.lax.rsqrt(var + 1e-5) * g + b


def _make_sc_msg(n_chunks):
    mesh = plsc.VectorSubcoreMesh(core_axis_name="c", subcore_axis_name="s")

    def body(g4, src1, dstr, combf, tvec, zr, acc_out,
             gsrc0, gsrc1, buf0, buf1, cbuf0, cbuf1, sbuf, didx, tv,
             gsem, ssem, acc):
        c = lax.axis_index("c")
        s = lax.axis_index("s")
        pltpu.sync_copy(tvec, tv)
        tvv = tv[...]
        slots = ((gsrc0, buf0, cbuf0), (gsrc1, buf1, cbuf1))

        for p in range(2):
            q = 2 * c + p
            pltpu.sync_copy(zr.at[pl.ds(s * RZ, RZ)], acc.at[pl.ds(s * RZ, RZ)])
            plsc.subcore_barrier()

            def load_and_fire(ci, sl):
                gs, buf, cb = slots[sl]
                g = s * n_chunks + ci
                pltpu.sync_copy(src1.at[pl.ds(g * K, K)], gs)
                pltpu.sync_copy(combf.at[q].at[pl.ds(g * 32, 32)], cb)
                pltpu.async_copy(g4.at[q].at[gs], buf, gsem)

            def do_chunk(ci, sl):
                gs, buf, cb = slots[sl]
                pltpu.make_async_copy(g4.at[q].at[gs], buf, gsem).wait()

                @pl.when(ci + 1 < n_chunks)
                def _():
                    load_and_fire(ci + 1, 1 - sl)

                @plsc.parallel_loop(0, K, unroll=8)
                def cbody(e):
                    for cc in range(2):
                        cs = pl.ds(cc * 16, 16)
                        a = buf[e, cs]
                        b = cb[cs]
                        m = jnp.maximum(a + b, 0.0) + EPS
                        ex = jnp.exp(m * tvv)
                        sbuf[e, cs] = m * ex
                        sbuf[e, pl.ds(32 + cc * 16, 16)] = ex

                base4 = (s * n_chunks + ci) * (K // 128)
                pltpu.sync_copy(dstr.at[pl.ds(base4, K // 128)], didx)
                descs = []
                for j in range(K // 128):
                    descs.append(pltpu.async_copy(
                        sbuf.at[pl.ds(j * 128, 128)],
                        acc.at[didx.at[j]], ssem, add=True))
                for dsc in descs:
                    dsc.wait()

            load_and_fire(0, 0)

            def pair(i, carry):
                do_chunk(2 * i, 0)
                do_chunk(2 * i + 1, 1)
                return carry

            lax.fori_loop(0, n_chunks // 2, pair, 0)
            plsc.subcore_barrier()
            pltpu.sync_copy(acc.at[pl.ds(s * RZ, RZ)],
                            acc_out.at[q].at[pl.ds(s * RZ, RZ)])

    return pl.kernel(
        body,
        out_type=jax.ShapeDtypeStruct((4, NPAD, HD), jnp.float32),
        mesh=mesh,
        compiler_params=pltpu.CompilerParams(use_tc_tiling_on_sc=False),
        scratch_types=[
            pltpu.VMEM((K,), jnp.int32),
            pltpu.VMEM((K,), jnp.int32),
            pltpu.VMEM((K, 32), jnp.float32),
            pltpu.VMEM((K, 32), jnp.float32),
            pltpu.VMEM((32,), jnp.float32),
            pltpu.VMEM((32,), jnp.float32),
            pltpu.VMEM((K, HD), jnp.float32),
            pltpu.VMEM((K // 128, 128), jnp.int32),
            pltpu.VMEM((16,), jnp.float32),
            pltpu.SemaphoreType.DMA,
            pltpu.SemaphoreType.DMA,
            pltpu.VMEM_SHARED((NPAD, HD), jnp.float32),
        ],
    )


def _embed_body(xr, xe1r, xe2r, lngr, lnbr, h0r, har4):
    xv = xr[...]
    na = xe1r.shape[0]
    nc = xe2r.shape[0]
    oh1 = (xv[:, 0:1] == lax.broadcasted_iota(jnp.int32, (xv.shape[0], na), 1)
           ).astype(jnp.float32)
    oh2 = (xv[:, 1:2] == lax.broadcasted_iota(jnp.int32, (xv.shape[0], nc), 1)
           ).astype(jnp.float32)
    h0 = (jnp.dot(oh1, xe1r[...], preferred_element_type=jnp.float32)
          + jnp.dot(oh2, xe2r[...], preferred_element_type=jnp.float32))
    h0r[...] = h0
    ha = jax.nn.relu(_ln(h0, lngr[...], lnbr[...]))
    for q in range(4):
        har4[q, :, :] = ha[:, 32 * q:32 * q + 32]


def _update_body(hr, har4, accr, w1r, b1r, gr, blnr, w2r, b2r,
                 lngr, lnbr, hnr, hanr4):
    num = jnp.concatenate([accr[q, :, :32] for q in range(4)], axis=-1)
    den = jnp.concatenate([accr[q, :, 32:] for q in range(4)], axis=-1)
    ha = jnp.concatenate([har4[q] for q in range(4)], axis=-1)
    out = num / (den + 1e-16) + ha
    z = jnp.dot(out, w1r[...], preferred_element_type=jnp.float32) + b1r[...]
    z = jax.nn.relu(_ln(z, gr[...], blnr[...]))
    m = jnp.dot(z, w2r[...], preferred_element_type=jnp.float32) + b2r[...]
    hn = hr[...] + m
    hnr[...] = hn
    han = jax.nn.relu(_ln(hn, lngr[...], lnbr[...]))
    for q in range(4):
        hanr4[q, :, :] = han[:, 32 * q:32 * q + 32]


def _pool_body(hr, br, fwr, fbr, w1r, b1r, w2r, b2r, outr):
    ng = 64
    oh = (br[...] == lax.broadcasted_iota(jnp.int32, (N, ng), 1)
          ).astype(jnp.float32)
    sums = lax.dot_general(oh, hr[...], (((0,), (0,)), ((), ())),
                           preferred_element_type=jnp.float32)
    cnts = jnp.sum(oh, axis=0)[:, None]
    pooled = sums / jnp.maximum(cnts, 1.0)
    f = jnp.dot(pooled, fwr[...], preferred_element_type=jnp.float32) + fbr[...]
    o = jax.nn.relu(
        jnp.dot(f, w1r[...], preferred_element_type=jnp.float32) + b1r[...])
    outr[...] = jnp.dot(o, w2r[...], preferred_element_type=jnp.float32) + b2r[...]


def _full(shape):
    return pl.BlockSpec(shape, lambda i: tuple(0 for _ in shape))


def kernel(x, edge_index, edge_attr, batch, xe1, xe2, ee1, ee2, t,
           mlp_w1, mlp_b1, mlp_g, mlp_bln, mlp_w2, mlp_b2, ln_g, ln_b,
           feat_w, feat_b, p_w1, p_b1, p_w2, p_b2):
    assert x.shape == (N, 2) and xe1.shape[1] == D
    L = ee1.shape[0]
    nbd = ee2.shape[1]
    ncomb = ee1.shape[1] * nbd
    E = edge_index.shape[1]
    EP = E + N
    per2 = 2 * NSUB * K
    EPAD = -(-(EP + ncomb * (K - 1)) // per2) * per2
    n_chunks = EPAD // (NSUB * K)
    TCH = EPAD // K

    i32 = edge_index.dtype
    sl = jnp.arange(N, dtype=i32)
    src0 = jnp.concatenate([edge_index[0], sl])
    dst0 = jnp.concatenate([edge_index[1], sl])
    ec0 = jnp.concatenate([edge_attr[:, 0] * nbd + edge_attr[:, 1],
                           jnp.full((N,), 4 * nbd, dtype=i32)])
    packed = src0 * jnp.int32(16384) + dst0
    _, s_pk = lax.sort([ec0, packed], num_keys=1)
    cnt = jnp.sum(
        (ec0[:, None] == jnp.arange(ncomb, dtype=i32)[None, :]).astype(i32),
        axis=0)
    pc = -(-cnt // K) * K
    cum_pc = jnp.cumsum(pc)
    offs = cum_pc - pc
    starts = jnp.cumsum(cnt) - cnt
    i = jnp.arange(EPAD, dtype=i32)
    b = jnp.clip(jnp.sum((i[:, None] >= cum_pc[None, :]).astype(i32), axis=1),
                 0, ncomb - 1)
    r = i - offs[b]
    valid = r < cnt[b]
    pk = s_pk[jnp.clip(starts[b] + r, 0, EP - 1)]
    src = jnp.where(valid, pk // 16384, i % N)
    dst = jnp.where(valid, pk % 16384,
                    N + i % (NPAD - N)).reshape(-1, 128)
    chunk_ec = b[jnp.arange(TCH) * K]

    comb = (ee1[:, :, None, :] + ee2[:, None, :, :]).reshape(L, ncomb, D)
    zr = jnp.zeros((NPAD, HD), jnp.float32)

    sc_msg = _make_sc_msg(n_chunks)

    R = 1000
    grid = (N // R,)
    embed = pl.pallas_call(
        _embed_body,
        grid=grid,
        in_specs=[pl.BlockSpec((R, 2), lambda i: (i, 0)),
                  _full(xe1.shape), _full(xe2.shape),
                  _full((1, D)), _full((1, D))],
        out_specs=[pl.BlockSpec((R, D), lambda i: (i, 0)),
                   pl.BlockSpec((4, R, 32), lambda i: (0, i, 0))],
        out_shape=[jax.ShapeDtypeStruct((N, D), jnp.float32),
                   jax.ShapeDtypeStruct((4, N, 32), jnp.float32)],
    )
    update = pl.pallas_call(
        _update_body,
        grid=grid,
        in_specs=[pl.BlockSpec((R, D), lambda i: (i, 0)),
                  pl.BlockSpec((4, R, 32), lambda i: (0, i, 0)),
                  pl.BlockSpec((4, R, HD), lambda i: (0, i, 0)),
                  _full((D, 2 * D)), _full((1, 2 * D)), _full((1, 2 * D)),
                  _full((1, 2 * D)), _full((2 * D, D)), _full((1, D)),
                  _full((1, D)), _full((1, D))],
        out_specs=[pl.BlockSpec((R, D), lambda i: (i, 0)),
                   pl.BlockSpec((4, R, 32), lambda i: (0, i, 0))],
        out_shape=[jax.ShapeDtypeStruct((N, D), jnp.float32),
                   jax.ShapeDtypeStruct((4, N, 32), jnp.float32)],
    )
    pool = pl.pallas_call(
        _pool_body,
        out_shape=jax.ShapeDtypeStruct((64, 2), jnp.float32),
    )

    crows = comb[:, chunk_ec, :]
    combf_all = jnp.stack([crows[:, :, 32 * q:32 * q + 32].reshape(L, -1)
                           for q in range(4)], axis=1)
    tvecs = jnp.broadcast_to(t[:, None].astype(jnp.float32), (L, 16))

    h, g4 = embed(x, xe1, xe2, ln_g[0][None], ln_b[0][None])
    for l in range(L):
        acc = sc_msg(g4, src, dst, combf_all[l], tvecs[l], zr)
        ln_n = (l + 1) % L
        h, g4 = update(h, g4, acc,
                       mlp_w1[l], mlp_b1[l][None], mlp_g[l][None],
                       mlp_bln[l][None], mlp_w2[l], mlp_b2[l][None],
                       ln_g[ln_n][None], ln_b[ln_n][None])
    return pool(h, batch[:, None], feat_w, feat_b[None],
                p_w1, p_b1[None], p_w2, p_b2[None])
